# baseline (speedup 1.0000x reference)
import jax
import jax.numpy as jnp
import numpy as np
from jax.experimental import pallas as pl
from jax.experimental.pallas import tpu as pltpu

INPUT_DIM = 8
D_MODEL = 64
N_HEADS = 8
HEAD_DIM = D_MODEL // N_HEADS
NUM_LAYERS = 2
DIM_FF = 256
EPS = 1e-5
NEG_INF = -1e9
B_TILE = 16
S = 32
MH = N_HEADS * S


def _enc_kernel(x_ref, pos_ref, wp_ref, bp_ref, wqv_ref, bqv_ref, wk_ref,
                bkt_ref, wo_ref, vec_ref, ffw1_ref, ffb1_ref, ffw2_ref,
                out_ref):
    bt, s, f = x_ref.shape
    m = bt * s
    d = D_MODEL

    x2 = x_ref[...].reshape(m, f)

    nz = jax.lax.dot_general(
        jnp.ones((1, f), jnp.float32), (x2 != 0.0).astype(jnp.float32),
        (((1,), (1,)), ((), ())), preferred_element_type=jnp.float32)
    col_j = jax.lax.broadcasted_iota(jnp.int32, (1, m), 1) % s
    pad_bias = jnp.where((nz == 0.0) & (col_j > 0), NEG_INF, 0.0)

    krow_h = jax.lax.broadcasted_iota(jnp.int32, (d, MH), 0) // HEAD_DIM
    kcol_h = jax.lax.broadcasted_iota(jnp.int32, (d, MH), 1) // S
    kmask = krow_h == kcol_h
    vrow_h = jax.lax.broadcasted_iota(jnp.int32, (MH, d), 0) // S
    vcol_h = jax.lax.broadcasted_iota(jnp.int32, (MH, d), 1) // HEAD_DIM
    vmask = vrow_h == vcol_h
    hm_bf = jnp.where(vmask, 1.0, 0.0).astype(jnp.bfloat16)
    e_row = jax.lax.broadcasted_iota(jnp.int32, (S, MH), 0)
    e_col = jax.lax.broadcasted_iota(jnp.int32, (S, MH), 1) % S
    e_bf = jnp.where(e_row == e_col, 1.0, 0.0).astype(jnp.bfloat16)

    h = (jnp.dot(x2.astype(jnp.bfloat16), wp_ref[...],
                 preferred_element_type=jnp.float32) + bp_ref[...])
    h = (h.reshape(bt, s, d) + pos_ref[...]).reshape(m, d)

    def layer_norm(z, g, b):
        mu = jnp.mean(z, axis=-1, keepdims=True)
        var = jnp.mean(jnp.square(z - mu), axis=-1, keepdims=True)
        return (z - mu) * jax.lax.rsqrt(var + EPS) * g + b

    for l in range(NUM_LAYERS):
        vec = vec_ref[l]
        bo, g1, be1 = vec[0:1], vec[1:2], vec[2:3]
        b2, g2, be2 = vec[3:4], vec[4:5], vec[5:6]

        hb = h.astype(jnp.bfloat16)
        qv = (jnp.dot(hb, wqv_ref[l], preferred_element_type=jnp.float32)
              + bqv_ref[l])
        q_bf = qv[:, :d].astype(jnp.bfloat16)
        v_bf = qv[:, d:].astype(jnp.bfloat16)
        kt = jax.lax.dot_general(
            wk_ref[l], hb, (((0,), (1,)), ((), ())),
            preferred_element_type=jnp.float32) + bkt_ref[l]
        kt_bf = kt.astype(jnp.bfloat16)

        outs = []
        for b in range(bt):
            sl = slice(S * b, S * (b + 1))
            krhs = jax.lax.dot_general(
                kt_bf[:, sl], e_bf, (((1,), (0,)), ((), ())),
                preferred_element_type=jnp.float32)
            krhs_bf = jnp.where(kmask, krhs, 0.0).astype(jnp.bfloat16)
            logits = jnp.dot(q_bf[sl, :], krhs_bf,
                             preferred_element_type=jnp.float32)
            pad_t = jnp.concatenate([pad_bias[:, sl]] * N_HEADS, axis=1)
            logits = logits + pad_t
            mx = jnp.max(logits, axis=-1, keepdims=True)
            p_bf = jnp.exp(logits - mx).astype(jnp.bfloat16)
            den = jnp.dot(p_bf, hm_bf,
                          preferred_element_type=jnp.float32)
            vtile = jnp.concatenate([v_bf[sl, :]] * N_HEADS, axis=0)
            vrhs = jnp.where(vmask, vtile, jnp.bfloat16(0.0))
            av = jnp.dot(p_bf, vrhs,
                         preferred_element_type=jnp.float32)
            outs.append(av / den)
        attn = jnp.concatenate(outs, axis=0)

        proj = jnp.dot(attn.astype(jnp.bfloat16), wo_ref[l],
                       preferred_element_type=jnp.float32) + bo
        y = layer_norm(h + proj, g1, be1)

        ff = jnp.maximum(
            jnp.dot(y.astype(jnp.bfloat16), ffw1_ref[l],
                    preferred_element_type=jnp.float32) + ffb1_ref[l], 0.0)
        ff2 = jnp.dot(ff.astype(jnp.bfloat16), ffw2_ref[l],
                      preferred_element_type=jnp.float32) + b2
        h = layer_norm(y + ff2, g2, be2)

    out_ref[...] = h.reshape(bt, s, d)[:, 0, :]


def _const_spec(shape):
    n = len(shape)
    return pl.BlockSpec(shape, lambda g, _n=n: (0,) * _n)


def kernel(x, cls_token, wp_t, bp, pos_embedding, qkvo_w, layer_vec,
           ff_b1, ff_w1, ff_w2):
    B, seq_len, F = x.shape
    s = seq_len + 1
    scale = np.float32(1.0 / np.sqrt(HEAD_DIM))

    wq, wk, wv, wo = (qkvo_w[:, i] for i in range(4))
    bq, bk, bv = (layer_vec[:, i] for i in range(3))
    wqv = jnp.concatenate([wq * scale, wv], axis=2).astype(jnp.bfloat16)
    bqv = jnp.concatenate([bq * scale, bv], axis=1)[:, None, :]
    wk_bf = wk.astype(jnp.bfloat16)
    bkt = bk[:, :, None]
    wo_bf = wo.astype(jnp.bfloat16)
    vec6 = layer_vec[:, 3:9]
    wp_bf = wp_t.astype(jnp.bfloat16)
    ffw1_bf = ff_w1.astype(jnp.bfloat16)
    ffw2_bf = ff_w2.astype(jnp.bfloat16)
    pos = pos_embedding[0, :s, :]

    cls = jnp.broadcast_to(cls_token, (B, 1, F))
    x_cat = jnp.concatenate([cls, x], axis=1)
    B_pad = ((B + B_TILE - 1) // B_TILE) * B_TILE
    if B_pad != B:
        x_cat = jnp.concatenate(
            [x_cat, jnp.zeros((B_pad - B, s, F), x_cat.dtype)], axis=0)

    weights = [pos, wp_bf, bp, wqv, bqv, wk_bf, bkt, wo_bf, vec6,
               ffw1_bf, ff_b1, ffw2_bf]
    grid = (B_pad // B_TILE,)
    in_specs = ([pl.BlockSpec((B_TILE, s, F), lambda g: (g, 0, 0))]
                + [_const_spec(tuple(w.shape)) for w in weights])
    out_spec = pl.BlockSpec((B_TILE, D_MODEL), lambda g: (g, 0))

    cls_hidden = pl.pallas_call(
        _enc_kernel,
        out_shape=jax.ShapeDtypeStruct((B_pad, D_MODEL), jnp.float32),
        grid=grid,
        in_specs=in_specs,
        out_specs=out_spec,
        compiler_params=pltpu.CompilerParams(
            dimension_semantics=("parallel",)),
    )(x_cat, *weights)

    return cls_hidden[:B, :]


# speedup vs baseline: 2.4704x; 2.4704x over previous
import jax
import jax.numpy as jnp
import numpy as np
from jax.experimental import pallas as pl
from jax.experimental.pallas import tpu as pltpu

INPUT_DIM = 8
D_MODEL = 64
N_HEADS = 8
HEAD_DIM = D_MODEL // N_HEADS
NUM_LAYERS = 2
DIM_FF = 256
EPS = 1e-5
NEG_INF = -1e9
B_TILE = 16
S = 32
MH = N_HEADS * S


def _enc_kernel(x_ref, pos_ref, wp_ref, bp_ref, wqv_ref, bqv_ref, wk_ref,
                bkt_ref, wo_ref, vec_ref, ffw1_ref, ffb1_ref, ffw2_ref,
                out_ref):
    bt, s, f = x_ref.shape
    m = bt * s
    d = D_MODEL

    x2 = x_ref[...].reshape(m, f)

    nz = jax.lax.dot_general(
        jnp.ones((1, f), jnp.float32), (x2 != 0.0).astype(jnp.float32),
        (((1,), (1,)), ((), ())), preferred_element_type=jnp.float32)
    col_j = jax.lax.broadcasted_iota(jnp.int32, (1, m), 1) % s
    pad_bias = jnp.where((nz == 0.0) & (col_j > 0), NEG_INF, 0.0)

    krow_h = jax.lax.broadcasted_iota(jnp.int32, (d, MH), 0) // HEAD_DIM
    kcol_h = jax.lax.broadcasted_iota(jnp.int32, (d, MH), 1) // S
    kmask = krow_h == kcol_h
    vrow_h = jax.lax.broadcasted_iota(jnp.int32, (MH, d), 0) // S
    vcol_h = jax.lax.broadcasted_iota(jnp.int32, (MH, d), 1) // HEAD_DIM
    vmask = vrow_h == vcol_h
    hm_bf = jnp.where(vmask, 1.0, 0.0).astype(jnp.bfloat16)

    h = (jnp.dot(x2.astype(jnp.bfloat16), wp_ref[...],
                 preferred_element_type=jnp.float32) + bp_ref[...])
    h = (h.reshape(bt, s, d) + pos_ref[...]).reshape(m, d)

    def layer_norm(z, g, b):
        mu = jnp.mean(z, axis=-1, keepdims=True)
        var = jnp.mean(jnp.square(z - mu), axis=-1, keepdims=True)
        return (z - mu) * jax.lax.rsqrt(var + EPS) * g + b

    for l in range(NUM_LAYERS):
        vec = vec_ref[l]
        bo, g1, be1 = vec[0:1], vec[1:2], vec[2:3]
        b2, g2, be2 = vec[3:4], vec[4:5], vec[5:6]

        hb = h.astype(jnp.bfloat16)
        qv = (jnp.dot(hb, wqv_ref[l], preferred_element_type=jnp.float32)
              + bqv_ref[l])
        q_bf = qv[:, :d].astype(jnp.bfloat16)
        v_bf = qv[:, d:].astype(jnp.bfloat16)
        kt = jax.lax.dot_general(
            wk_ref[l], hb, (((0,), (1,)), ((), ())),
            preferred_element_type=jnp.float32) + bkt_ref[l]
        kt_bf = kt.astype(jnp.bfloat16)

        logits_l = [None] * bt
        p_l = [None] * bt
        vrhs_l = [None] * bt
        den_l = [None] * bt
        av_l = [None] * bt
        outs = [None] * bt
        for i in range(bt + 3):
            if i < bt:
                b = i
                sl = slice(S * b, S * (b + 1))
                ktile = jnp.concatenate([kt_bf[:, sl]] * N_HEADS, axis=1)
                krhs_bf = jnp.where(kmask, ktile, jnp.bfloat16(0.0))
                lg = jnp.dot(q_bf[sl, :], krhs_bf,
                             preferred_element_type=jnp.float32)
                pad_t = jnp.concatenate([pad_bias[:, sl]] * N_HEADS,
                                        axis=1)
                logits_l[b] = lg + pad_t
                vtile = jnp.concatenate([v_bf[sl, :]] * N_HEADS, axis=0)
                vrhs_l[b] = jnp.where(vmask, vtile, jnp.bfloat16(0.0))
            if 1 <= i < bt + 1:
                b = i - 1
                lg = logits_l[b]
                mx = jnp.max(lg, axis=-1, keepdims=True)
                p_l[b] = jnp.exp(lg - mx).astype(jnp.bfloat16)
            if 2 <= i < bt + 2:
                b = i - 2
                den_l[b] = jnp.dot(p_l[b], hm_bf,
                                   preferred_element_type=jnp.float32)
                av_l[b] = jnp.dot(p_l[b], vrhs_l[b],
                                  preferred_element_type=jnp.float32)
            if 3 <= i:
                b = i - 3
                outs[b] = av_l[b] / den_l[b]
        attn = jnp.concatenate(outs, axis=0)

        proj = jnp.dot(attn.astype(jnp.bfloat16), wo_ref[l],
                       preferred_element_type=jnp.float32) + bo
        y = layer_norm(h + proj, g1, be1)

        ff = jnp.maximum(
            jnp.dot(y.astype(jnp.bfloat16), ffw1_ref[l],
                    preferred_element_type=jnp.float32) + ffb1_ref[l], 0.0)
        ff2 = jnp.dot(ff.astype(jnp.bfloat16), ffw2_ref[l],
                      preferred_element_type=jnp.float32) + b2
        h = layer_norm(y + ff2, g2, be2)

    out_ref[...] = h.reshape(bt, s, d)[:, 0, :]


def _const_spec(shape):
    n = len(shape)
    return pl.BlockSpec(shape, lambda g, _n=n: (0,) * _n)


def kernel(x, cls_token, wp_t, bp, pos_embedding, qkvo_w, layer_vec,
           ff_b1, ff_w1, ff_w2):
    B, seq_len, F = x.shape
    s = seq_len + 1
    scale = np.float32(1.0 / np.sqrt(HEAD_DIM))

    wq, wk, wv, wo = (qkvo_w[:, i] for i in range(4))
    bq, bk, bv = (layer_vec[:, i] for i in range(3))
    wqv = jnp.concatenate([wq * scale, wv], axis=2).astype(jnp.bfloat16)
    bqv = jnp.concatenate([bq * scale, bv], axis=1)[:, None, :]
    wk_bf = wk.astype(jnp.bfloat16)
    bkt = bk[:, :, None]
    wo_bf = wo.astype(jnp.bfloat16)
    vec6 = layer_vec[:, 3:9]
    wp_bf = wp_t.astype(jnp.bfloat16)
    ffw1_bf = ff_w1.astype(jnp.bfloat16)
    ffw2_bf = ff_w2.astype(jnp.bfloat16)
    pos = pos_embedding[0, :s, :]

    cls = jnp.broadcast_to(cls_token, (B, 1, F))
    x_cat = jnp.concatenate([cls, x], axis=1)
    B_pad = ((B + B_TILE - 1) // B_TILE) * B_TILE
    if B_pad != B:
        x_cat = jnp.concatenate(
            [x_cat, jnp.zeros((B_pad - B, s, F), x_cat.dtype)], axis=0)

    weights = [pos, wp_bf, bp, wqv, bqv, wk_bf, bkt, wo_bf, vec6,
               ffw1_bf, ff_b1, ffw2_bf]
    grid = (B_pad // B_TILE,)
    in_specs = ([pl.BlockSpec((B_TILE, s, F), lambda g: (g, 0, 0))]
                + [_const_spec(tuple(w.shape)) for w in weights])
    out_spec = pl.BlockSpec((B_TILE, D_MODEL), lambda g: (g, 0))

    cls_hidden = pl.pallas_call(
        _enc_kernel,
        out_shape=jax.ShapeDtypeStruct((B_pad, D_MODEL), jnp.float32),
        grid=grid,
        in_specs=in_specs,
        out_specs=out_spec,
        compiler_params=pltpu.CompilerParams(
            dimension_semantics=("parallel",)),
    )(x_cat, *weights)

    return cls_hidden[:B, :]


# speedup vs baseline: 3.5435x; 1.4344x over previous
import jax
import jax.numpy as jnp
import numpy as np
from jax.experimental import pallas as pl
from jax.experimental.pallas import tpu as pltpu

INPUT_DIM = 8
D_MODEL = 64
N_HEADS = 8
HEAD_DIM = D_MODEL // N_HEADS
NUM_LAYERS = 2
DIM_FF = 256
EPS = 1e-5
NEG_INF = -1e9
B_TILE = 64
S = 32
MH = N_HEADS * S


def _enc_kernel(x_ref, pos_ref, wp_ref, bp_ref, wqv_ref, bqv_ref, wk_ref,
                bkt_ref, wo_ref, vec_ref, ffw1_ref, ffb1_ref, ffw2_ref,
                out_ref):
    bt, s, f = x_ref.shape
    m = bt * s
    d = D_MODEL

    x2 = x_ref[...].reshape(m, f)

    nz = jax.lax.dot_general(
        jnp.ones((1, f), jnp.float32), (x2 != 0.0).astype(jnp.float32),
        (((1,), (1,)), ((), ())), preferred_element_type=jnp.float32)
    col_j = jax.lax.broadcasted_iota(jnp.int32, (1, m), 1) % s
    pad_bias = jnp.where((nz == 0.0) & (col_j > 0), NEG_INF, 0.0)

    krow_h = jax.lax.broadcasted_iota(jnp.int32, (d, MH), 0) // HEAD_DIM
    kcol_h = jax.lax.broadcasted_iota(jnp.int32, (d, MH), 1) // S
    kmask = krow_h == kcol_h
    vrow_h = jax.lax.broadcasted_iota(jnp.int32, (MH, d), 0) // S
    vcol_h = jax.lax.broadcasted_iota(jnp.int32, (MH, d), 1) // HEAD_DIM
    vmask = vrow_h == vcol_h
    hm_bf = jnp.where(vmask, 1.0, 0.0).astype(jnp.bfloat16)

    h = (jnp.dot(x2.astype(jnp.bfloat16), wp_ref[...],
                 preferred_element_type=jnp.float32) + bp_ref[...])
    h = (h.reshape(bt, s, d) + pos_ref[...]).reshape(m, d)

    def layer_norm(z, g, b):
        mu = jnp.mean(z, axis=-1, keepdims=True)
        var = jnp.mean(jnp.square(z - mu), axis=-1, keepdims=True)
        return (z - mu) * jax.lax.rsqrt(var + EPS) * g + b

    for l in range(NUM_LAYERS):
        vec = vec_ref[l]
        bo, g1, be1 = vec[0:1], vec[1:2], vec[2:3]
        b2, g2, be2 = vec[3:4], vec[4:5], vec[5:6]

        hb = h.astype(jnp.bfloat16)
        qv = (jnp.dot(hb, wqv_ref[l], preferred_element_type=jnp.float32)
              + bqv_ref[l])
        q_bf = qv[:, :d].astype(jnp.bfloat16)
        v_bf = qv[:, d:].astype(jnp.bfloat16)
        kt = jax.lax.dot_general(
            wk_ref[l], hb, (((0,), (1,)), ((), ())),
            preferred_element_type=jnp.float32) + bkt_ref[l]
        kt_bf = kt.astype(jnp.bfloat16)

        logits_l = [None] * bt
        p_l = [None] * bt
        vrhs_l = [None] * bt
        den_l = [None] * bt
        av_l = [None] * bt
        outs = [None] * bt
        for i in range(bt + 3):
            if i < bt:
                b = i
                sl = slice(S * b, S * (b + 1))
                ktile = jnp.concatenate([kt_bf[:, sl]] * N_HEADS, axis=1)
                krhs_bf = jnp.where(kmask, ktile, jnp.bfloat16(0.0))
                lg = jnp.dot(q_bf[sl, :], krhs_bf,
                             preferred_element_type=jnp.float32)
                pad_t = jnp.concatenate([pad_bias[:, sl]] * N_HEADS,
                                        axis=1)
                logits_l[b] = lg + pad_t
                vtile = jnp.concatenate([v_bf[sl, :]] * N_HEADS, axis=0)
                vrhs_l[b] = jnp.where(vmask, vtile, jnp.bfloat16(0.0))
            if 1 <= i < bt + 1:
                b = i - 1
                lg = logits_l[b]
                mx = jnp.max(lg, axis=-1, keepdims=True)
                p_l[b] = jnp.exp(lg - mx).astype(jnp.bfloat16)
            if 2 <= i < bt + 2:
                b = i - 2
                den_l[b] = jnp.dot(p_l[b], hm_bf,
                                   preferred_element_type=jnp.float32)
                av_l[b] = jnp.dot(p_l[b], vrhs_l[b],
                                  preferred_element_type=jnp.float32)
            if 3 <= i:
                b = i - 3
                outs[b] = av_l[b] / den_l[b]
        attn = jnp.concatenate(outs, axis=0)

        proj = jnp.dot(attn.astype(jnp.bfloat16), wo_ref[l],
                       preferred_element_type=jnp.float32) + bo
        y = layer_norm(h + proj, g1, be1)

        ff = jnp.maximum(
            jnp.dot(y.astype(jnp.bfloat16), ffw1_ref[l],
                    preferred_element_type=jnp.float32) + ffb1_ref[l], 0.0)
        ff2 = jnp.dot(ff.astype(jnp.bfloat16), ffw2_ref[l],
                      preferred_element_type=jnp.float32) + b2
        h = layer_norm(y + ff2, g2, be2)

    out_ref[...] = h.reshape(bt, s, d)[:, 0, :]


def _const_spec(shape):
    n = len(shape)
    return pl.BlockSpec(shape, lambda g, _n=n: (0,) * _n)


def kernel(x, cls_token, wp_t, bp, pos_embedding, qkvo_w, layer_vec,
           ff_b1, ff_w1, ff_w2):
    B, seq_len, F = x.shape
    s = seq_len + 1
    scale = np.float32(1.0 / np.sqrt(HEAD_DIM))

    wq, wk, wv, wo = (qkvo_w[:, i] for i in range(4))
    bq, bk, bv = (layer_vec[:, i] for i in range(3))
    wqv = jnp.concatenate([wq * scale, wv], axis=2).astype(jnp.bfloat16)
    bqv = jnp.concatenate([bq * scale, bv], axis=1)[:, None, :]
    wk_bf = wk.astype(jnp.bfloat16)
    bkt = bk[:, :, None]
    wo_bf = wo.astype(jnp.bfloat16)
    vec6 = layer_vec[:, 3:9]
    wp_bf = wp_t.astype(jnp.bfloat16)
    ffw1_bf = ff_w1.astype(jnp.bfloat16)
    ffw2_bf = ff_w2.astype(jnp.bfloat16)
    pos = pos_embedding[0, :s, :]

    cls = jnp.broadcast_to(cls_token, (B, 1, F))
    x_cat = jnp.concatenate([cls, x], axis=1)
    B_pad = ((B + B_TILE - 1) // B_TILE) * B_TILE
    if B_pad != B:
        x_cat = jnp.concatenate(
            [x_cat, jnp.zeros((B_pad - B, s, F), x_cat.dtype)], axis=0)

    weights = [pos, wp_bf, bp, wqv, bqv, wk_bf, bkt, wo_bf, vec6,
               ffw1_bf, ff_b1, ffw2_bf]
    grid = (B_pad // B_TILE,)
    in_specs = ([pl.BlockSpec((B_TILE, s, F), lambda g: (g, 0, 0))]
                + [_const_spec(tuple(w.shape)) for w in weights])
    out_spec = pl.BlockSpec((B_TILE, D_MODEL), lambda g: (g, 0))

    cls_hidden = pl.pallas_call(
        _enc_kernel,
        out_shape=jax.ShapeDtypeStruct((B_pad, D_MODEL), jnp.float32),
        grid=grid,
        in_specs=in_specs,
        out_specs=out_spec,
        compiler_params=pltpu.CompilerParams(
            dimension_semantics=("parallel",)),
    )(x_cat, *weights)

    return cls_hidden[:B, :]


# speedup vs baseline: 3.6144x; 1.0200x over previous
import jax
import jax.numpy as jnp
import numpy as np
from jax.experimental import pallas as pl
from jax.experimental.pallas import tpu as pltpu

INPUT_DIM = 8
D_MODEL = 64
N_HEADS = 8
HEAD_DIM = D_MODEL // N_HEADS
NUM_LAYERS = 2
DIM_FF = 256
EPS = 1e-5
NEG_INF = -1e9
B_TILE = 128
S = 32
MH = N_HEADS * S


def _enc_kernel(x_ref, pos_ref, wp_ref, bp_ref, wqv_ref, bqv_ref, wk_ref,
                bkt_ref, wo_ref, vec_ref, ffw1_ref, ffb1_ref, ffw2_ref,
                out_ref):
    bt, s, f = x_ref.shape
    m = bt * s
    d = D_MODEL

    x2 = x_ref[...].reshape(m, f)

    nz = jax.lax.dot_general(
        jnp.ones((1, f), jnp.float32), (x2 != 0.0).astype(jnp.float32),
        (((1,), (1,)), ((), ())), preferred_element_type=jnp.float32)
    col_j = jax.lax.broadcasted_iota(jnp.int32, (1, m), 1) % s
    pad_bias = jnp.where((nz == 0.0) & (col_j > 0), NEG_INF, 0.0)

    krow_h = jax.lax.broadcasted_iota(jnp.int32, (d, MH), 0) // HEAD_DIM
    kcol_h = jax.lax.broadcasted_iota(jnp.int32, (d, MH), 1) // S
    kmask_bf = jnp.where(krow_h == kcol_h, 1.0, 0.0).astype(jnp.bfloat16)
    vrow_h = jax.lax.broadcasted_iota(jnp.int32, (MH, d), 0) // S
    vcol_h = jax.lax.broadcasted_iota(jnp.int32, (MH, d), 1) // HEAD_DIM
    hm_bf = jnp.where(vrow_h == vcol_h, 1.0, 0.0).astype(jnp.bfloat16)

    h = (jnp.dot(x2.astype(jnp.bfloat16), wp_ref[...],
                 preferred_element_type=jnp.float32) + bp_ref[...])
    h = (h.reshape(bt, s, d) + pos_ref[...]).reshape(m, d)

    def layer_norm(z, g, b):
        mu = jnp.mean(z, axis=-1, keepdims=True)
        var = jnp.mean(jnp.square(z - mu), axis=-1, keepdims=True)
        return (z - mu) * jax.lax.rsqrt(var + EPS) * g + b

    for l in range(NUM_LAYERS):
        vec = vec_ref[l]
        bo, g1, be1 = vec[0:1], vec[1:2], vec[2:3]
        b2, g2, be2 = vec[3:4], vec[4:5], vec[5:6]

        hb = h.astype(jnp.bfloat16)
        qv = (jnp.dot(hb, wqv_ref[l], preferred_element_type=jnp.float32)
              + bqv_ref[l])
        q_bf = qv[:, :d].astype(jnp.bfloat16)
        v_bf = qv[:, d:].astype(jnp.bfloat16)
        kt = jax.lax.dot_general(
            wk_ref[l], hb, (((0,), (1,)), ((), ())),
            preferred_element_type=jnp.float32) + bkt_ref[l]
        kt_bf = kt.astype(jnp.bfloat16)

        logits_l = [None] * bt
        p_l = [None] * bt
        vrhs_l = [None] * bt
        den_l = [None] * bt
        av_l = [None] * bt
        outs = [None] * bt
        for i in range(bt + 3):
            if i < bt:
                b = i
                sl = slice(S * b, S * (b + 1))
                ktile = jnp.concatenate([kt_bf[:, sl]] * N_HEADS, axis=1)
                krhs_bf = ktile * kmask_bf
                lg = jnp.dot(q_bf[sl, :], krhs_bf,
                             preferred_element_type=jnp.float32)
                pad_t = jnp.concatenate([pad_bias[:, sl]] * N_HEADS,
                                        axis=1)
                logits_l[b] = lg + pad_t
                vtile = jnp.concatenate([v_bf[sl, :]] * N_HEADS, axis=0)
                vrhs_l[b] = vtile * hm_bf
            if 1 <= i < bt + 1:
                b = i - 1
                lg = logits_l[b]
                mx = jnp.max(lg, axis=-1, keepdims=True)
                p_l[b] = jnp.exp(lg - mx).astype(jnp.bfloat16)
            if 2 <= i < bt + 2:
                b = i - 2
                den_l[b] = jnp.dot(p_l[b], hm_bf,
                                   preferred_element_type=jnp.float32)
                av_l[b] = jnp.dot(p_l[b], vrhs_l[b],
                                  preferred_element_type=jnp.float32)
            if 3 <= i:
                b = i - 3
                outs[b] = av_l[b] / den_l[b]
        attn = jnp.concatenate(outs, axis=0)

        proj = jnp.dot(attn.astype(jnp.bfloat16), wo_ref[l],
                       preferred_element_type=jnp.float32) + bo
        y = layer_norm(h + proj, g1, be1)

        ff = jnp.maximum(
            jnp.dot(y.astype(jnp.bfloat16), ffw1_ref[l],
                    preferred_element_type=jnp.float32) + ffb1_ref[l], 0.0)
        ff2 = jnp.dot(ff.astype(jnp.bfloat16), ffw2_ref[l],
                      preferred_element_type=jnp.float32) + b2
        h = layer_norm(y + ff2, g2, be2)

    out_ref[...] = h.reshape(bt, s, d)[:, 0, :]


def _const_spec(shape):
    n = len(shape)
    return pl.BlockSpec(shape, lambda g, _n=n: (0,) * _n)


def kernel(x, cls_token, wp_t, bp, pos_embedding, qkvo_w, layer_vec,
           ff_b1, ff_w1, ff_w2):
    B, seq_len, F = x.shape
    s = seq_len + 1
    scale = np.float32(1.0 / np.sqrt(HEAD_DIM))

    wq, wk, wv, wo = (qkvo_w[:, i] for i in range(4))
    bq, bk, bv = (layer_vec[:, i] for i in range(3))
    wqv = jnp.concatenate([wq * scale, wv], axis=2).astype(jnp.bfloat16)
    bqv = jnp.concatenate([bq * scale, bv], axis=1)[:, None, :]
    wk_bf = wk.astype(jnp.bfloat16)
    bkt = bk[:, :, None]
    wo_bf = wo.astype(jnp.bfloat16)
    vec6 = layer_vec[:, 3:9]
    wp_bf = wp_t.astype(jnp.bfloat16)
    ffw1_bf = ff_w1.astype(jnp.bfloat16)
    ffw2_bf = ff_w2.astype(jnp.bfloat16)
    pos = pos_embedding[0, :s, :]

    cls = jnp.broadcast_to(cls_token, (B, 1, F))
    x_cat = jnp.concatenate([cls, x], axis=1)
    B_pad = ((B + B_TILE - 1) // B_TILE) * B_TILE
    if B_pad != B:
        x_cat = jnp.concatenate(
            [x_cat, jnp.zeros((B_pad - B, s, F), x_cat.dtype)], axis=0)

    weights = [pos, wp_bf, bp, wqv, bqv, wk_bf, bkt, wo_bf, vec6,
               ffw1_bf, ff_b1, ffw2_bf]
    grid = (B_pad // B_TILE,)
    in_specs = ([pl.BlockSpec((B_TILE, s, F), lambda g: (g, 0, 0))]
                + [_const_spec(tuple(w.shape)) for w in weights])
    out_spec = pl.BlockSpec((B_TILE, D_MODEL), lambda g: (g, 0))

    cls_hidden = pl.pallas_call(
        _enc_kernel,
        out_shape=jax.ShapeDtypeStruct((B_pad, D_MODEL), jnp.float32),
        grid=grid,
        in_specs=in_specs,
        out_specs=out_spec,
        compiler_params=pltpu.CompilerParams(
            dimension_semantics=("parallel",)),
    )(x_cat, *weights)

    return cls_hidden[:B, :]


# speedup vs baseline: 4.5246x; 1.2518x over previous
import jax
import jax.numpy as jnp
import numpy as np
from jax.experimental import pallas as pl
from jax.experimental.pallas import tpu as pltpu

INPUT_DIM = 8
D_MODEL = 64
N_HEADS = 8
HEAD_DIM = D_MODEL // N_HEADS
NUM_LAYERS = 2
DIM_FF = 256
EPS = 1e-5
NEG_INF = -1e9
B_TILE = 64
S = 32
MH = N_HEADS * S


def _enc_kernel(x_ref, pos_ref, wp_ref, bp_ref, wqv_ref, bqv_ref, wk_ref,
                bkt_ref, wo_ref, vec_ref, ffw1_ref, ffb1_ref, ffw2_ref,
                out_ref, q_scr, v_scr, kt_scr, at_scr):
    bt, s, f = x_ref.shape
    m = bt * s
    d = D_MODEL

    x2 = x_ref[...].reshape(m, f)

    nz = jax.lax.dot_general(
        jnp.ones((1, f), jnp.float32), (x2 != 0.0).astype(jnp.float32),
        (((1,), (1,)), ((), ())), preferred_element_type=jnp.float32)
    col_j = jax.lax.broadcasted_iota(jnp.int32, (1, m), 1) % s
    pad_bias = jnp.where((nz == 0.0) & (col_j > 0), NEG_INF, 0.0)

    krow_h = jax.lax.broadcasted_iota(jnp.int32, (d, MH), 0) // HEAD_DIM
    kcol_h = jax.lax.broadcasted_iota(jnp.int32, (d, MH), 1) // S
    kmask_bf = jnp.where(krow_h == kcol_h, 1.0, 0.0).astype(jnp.bfloat16)
    vrow_h = jax.lax.broadcasted_iota(jnp.int32, (MH, d), 0) // S
    vcol_h = jax.lax.broadcasted_iota(jnp.int32, (MH, d), 1) // HEAD_DIM
    hm_bf = jnp.where(vrow_h == vcol_h, 1.0, 0.0).astype(jnp.bfloat16)

    h = (jnp.dot(x2.astype(jnp.bfloat16), wp_ref[...],
                 preferred_element_type=jnp.float32) + bp_ref[...])
    h = (h.reshape(bt, s, d) + pos_ref[...]).reshape(m, d)

    def layer_norm(z, g, b):
        mu = jnp.mean(z, axis=-1, keepdims=True)
        m2 = jnp.mean(jnp.square(z), axis=-1, keepdims=True)
        var = m2 - jnp.square(mu)
        return (z - mu) * jax.lax.rsqrt(var + EPS) * g + b

    for l in range(NUM_LAYERS):
        vec = vec_ref[l]
        bo, g1, be1 = vec[0:1], vec[1:2], vec[2:3]
        b2, g2, be2 = vec[3:4], vec[4:5], vec[5:6]

        hb = h.astype(jnp.bfloat16)
        qv = (jnp.dot(hb, wqv_ref[l], preferred_element_type=jnp.float32)
              + bqv_ref[l])
        q_scr[...] = qv[:, :d].astype(jnp.bfloat16)
        v_scr[...] = qv[:, d:].astype(jnp.bfloat16)
        kt = jax.lax.dot_general(
            wk_ref[l], hb, (((0,), (1,)), ((), ())),
            preferred_element_type=jnp.float32) + bkt_ref[l]
        kt_scr[...] = kt.astype(jnp.bfloat16)

        logits_l = [None] * bt
        p_l = [None] * bt
        vrhs_l = [None] * bt
        den_l = [None] * bt
        av_l = [None] * bt
        pad_l = [None] * bt
        D1, D2, D3 = 3, 5, 7
        for i in range(bt + D3):
            if i < bt:
                b = i
                sl = slice(S * b, S * (b + 1))
                ktile = jnp.concatenate([kt_scr[:, sl]] * N_HEADS, axis=1)
                krhs_bf = ktile * kmask_bf
                logits_l[b] = jnp.dot(q_scr[sl, :], krhs_bf,
                                      preferred_element_type=jnp.float32)
                pad_l[b] = jnp.concatenate([pad_bias[:, sl]] * N_HEADS,
                                           axis=1)
                vtile = jnp.concatenate([v_scr[sl, :]] * N_HEADS, axis=0)
                vrhs_l[b] = vtile * hm_bf
            if D1 <= i < bt + D1:
                b = i - D1
                lg = logits_l[b]
                mx = jnp.max(lg, axis=-1, keepdims=True)
                p_l[b] = jnp.exp(lg + (pad_l[b] - mx)).astype(jnp.bfloat16)
            if D2 <= i < bt + D2:
                b = i - D2
                den_l[b] = jnp.dot(p_l[b], hm_bf,
                                   preferred_element_type=jnp.float32)
                av_l[b] = jnp.dot(p_l[b], vrhs_l[b],
                                  preferred_element_type=jnp.float32)
            if D3 <= i:
                b = i - D3
                sl3 = slice(S * b, S * (b + 1))
                at_scr[sl3, :] = (av_l[b] / den_l[b]).astype(jnp.bfloat16)

        proj = jnp.dot(at_scr[...], wo_ref[l],
                       preferred_element_type=jnp.float32) + bo
        y = layer_norm(h + proj, g1, be1)

        ff = jnp.maximum(
            jnp.dot(y.astype(jnp.bfloat16), ffw1_ref[l],
                    preferred_element_type=jnp.float32) + ffb1_ref[l], 0.0)
        ff2 = jnp.dot(ff.astype(jnp.bfloat16), ffw2_ref[l],
                      preferred_element_type=jnp.float32) + b2
        h = layer_norm(y + ff2, g2, be2)

    out_ref[...] = h.reshape(bt, s, d)[:, 0, :]


def _const_spec(shape):
    n = len(shape)
    return pl.BlockSpec(shape, lambda g, _n=n: (0,) * _n)


def kernel(x, cls_token, wp_t, bp, pos_embedding, qkvo_w, layer_vec,
           ff_b1, ff_w1, ff_w2):
    B, seq_len, F = x.shape
    s = seq_len + 1
    scale = np.float32(1.0 / np.sqrt(HEAD_DIM))

    wq, wk, wv, wo = (qkvo_w[:, i] for i in range(4))
    bq, bk, bv = (layer_vec[:, i] for i in range(3))
    wqv = jnp.concatenate([wq * scale, wv], axis=2).astype(jnp.bfloat16)
    bqv = jnp.concatenate([bq * scale, bv], axis=1)[:, None, :]
    wk_bf = wk.astype(jnp.bfloat16)
    bkt = bk[:, :, None]
    wo_bf = wo.astype(jnp.bfloat16)
    vec6 = layer_vec[:, 3:9]
    wp_bf = wp_t.astype(jnp.bfloat16)
    ffw1_bf = ff_w1.astype(jnp.bfloat16)
    ffw2_bf = ff_w2.astype(jnp.bfloat16)
    pos = pos_embedding[0, :s, :]

    cls = jnp.broadcast_to(cls_token, (B, 1, F))
    x_cat = jnp.concatenate([cls, x], axis=1)
    B_pad = ((B + B_TILE - 1) // B_TILE) * B_TILE
    if B_pad != B:
        x_cat = jnp.concatenate(
            [x_cat, jnp.zeros((B_pad - B, s, F), x_cat.dtype)], axis=0)

    weights = [pos, wp_bf, bp, wqv, bqv, wk_bf, bkt, wo_bf, vec6,
               ffw1_bf, ff_b1, ffw2_bf]
    grid = (B_pad // B_TILE,)
    in_specs = ([pl.BlockSpec((B_TILE, s, F), lambda g: (g, 0, 0))]
                + [_const_spec(tuple(w.shape)) for w in weights])
    out_spec = pl.BlockSpec((B_TILE, D_MODEL), lambda g: (g, 0))

    m = B_TILE * s
    cls_hidden = pl.pallas_call(
        _enc_kernel,
        out_shape=jax.ShapeDtypeStruct((B_pad, D_MODEL), jnp.float32),
        grid=grid,
        in_specs=in_specs,
        out_specs=out_spec,
        scratch_shapes=[pltpu.VMEM((m, D_MODEL), jnp.bfloat16),
                        pltpu.VMEM((m, D_MODEL), jnp.bfloat16),
                        pltpu.VMEM((D_MODEL, m), jnp.bfloat16),
                        pltpu.VMEM((m, D_MODEL), jnp.bfloat16)],
        compiler_params=pltpu.CompilerParams(
            dimension_semantics=("parallel",)),
    )(x_cat, *weights)

    return cls_hidden[:B, :]


# speedup vs baseline: 5.6254x; 1.2433x over previous
import jax
import jax.numpy as jnp
import numpy as np
from jax.experimental import pallas as pl
from jax.experimental.pallas import tpu as pltpu

INPUT_DIM = 8
D_MODEL = 64
N_HEADS = 8
HEAD_DIM = D_MODEL // N_HEADS
NUM_LAYERS = 2
DIM_FF = 256
EPS = 1e-5
NEG_INF = -1e9
B_TILE = 64
S = 32
MH = N_HEADS * S


def _enc_kernel(x_ref, pos_ref, wp_ref, bp_ref, wqv_ref, bqv_ref, wk_ref,
                bkt_ref, wo_ref, vec_ref, ffw1_ref, ffb1_ref, ffw2_ref,
                out_ref, q_scr, v_scr, kt_scr, at_scr):
    bt, s, f = x_ref.shape
    m = bt * s
    d = D_MODEL

    x2 = x_ref[...].reshape(m, f)

    nz = jax.lax.dot_general(
        jnp.ones((1, f), jnp.float32), (x2 != 0.0).astype(jnp.float32),
        (((1,), (1,)), ((), ())), preferred_element_type=jnp.float32)
    col_j = jax.lax.broadcasted_iota(jnp.int32, (1, m), 1) % s
    pad_bias = jnp.where((nz == 0.0) & (col_j > 0), NEG_INF, 0.0)

    krow_h = jax.lax.broadcasted_iota(jnp.int32, (d, MH), 0) // HEAD_DIM
    kcol_h = jax.lax.broadcasted_iota(jnp.int32, (d, MH), 1) // S
    kmask_bf = jnp.where(krow_h == kcol_h, 1.0, 0.0).astype(jnp.bfloat16)
    vrow_h = jax.lax.broadcasted_iota(jnp.int32, (MH, d), 0) // S
    vcol_h = jax.lax.broadcasted_iota(jnp.int32, (MH, d), 1) // HEAD_DIM
    hm_bf = jnp.where(vrow_h == vcol_h, 1.0, 0.0).astype(jnp.bfloat16)

    h = (jnp.dot(x2.astype(jnp.bfloat16), wp_ref[...],
                 preferred_element_type=jnp.float32) + bp_ref[...])
    h = (h.reshape(bt, s, d) + pos_ref[...]).reshape(m, d)

    def layer_norm(z, g, b):
        mu = jnp.mean(z, axis=-1, keepdims=True)
        m2 = jnp.mean(jnp.square(z), axis=-1, keepdims=True)
        var = m2 - jnp.square(mu)
        return (z - mu) * jax.lax.rsqrt(var + EPS) * g + b

    for l in range(NUM_LAYERS):
        vec = vec_ref[l]
        bo, g1, be1 = vec[0:1], vec[1:2], vec[2:3]
        b2, g2, be2 = vec[3:4], vec[4:5], vec[5:6]

        hb = h.astype(jnp.bfloat16)
        qv = (jnp.dot(hb, wqv_ref[l], preferred_element_type=jnp.float32)
              + bqv_ref[l])
        q_scr[...] = qv[:, :d].astype(jnp.bfloat16)
        v_scr[...] = qv[:, d:].astype(jnp.bfloat16)
        kt = jax.lax.dot_general(
            wk_ref[l], hb, (((0,), (1,)), ((), ())),
            preferred_element_type=jnp.float32) + bkt_ref[l]
        kt_scr[...] = kt.astype(jnp.bfloat16)

        logits_l = [None] * bt
        p_l = [None] * bt
        vrhs_l = [None] * bt
        den_l = [None] * bt
        av_l = [None] * bt
        pad_l = [None] * bt
        D1, D2, D3 = 6, 10, 13
        for i in range(bt + D3):
            if i < bt:
                b = i
                sl = slice(S * b, S * (b + 1))
                ktile = jnp.concatenate([kt_scr[:, sl]] * N_HEADS, axis=1)
                krhs_bf = ktile * kmask_bf
                logits_l[b] = jnp.dot(q_scr[sl, :], krhs_bf,
                                      preferred_element_type=jnp.float32)
                pad_l[b] = jnp.concatenate([pad_bias[:, sl]] * N_HEADS,
                                           axis=1)
                vtile = jnp.concatenate([v_scr[sl, :]] * N_HEADS, axis=0)
                vrhs_l[b] = vtile * hm_bf
            if D1 <= i < bt + D1:
                b = i - D1
                lg = logits_l[b]
                mx = jnp.max(lg, axis=-1, keepdims=True)
                p_l[b] = jnp.exp(lg + (pad_l[b] - mx)).astype(jnp.bfloat16)
            if D2 <= i < bt + D2:
                b = i - D2
                den_l[b] = jnp.dot(p_l[b], hm_bf,
                                   preferred_element_type=jnp.float32)
                av_l[b] = jnp.dot(p_l[b], vrhs_l[b],
                                  preferred_element_type=jnp.float32)
            if D3 <= i:
                b = i - D3
                sl3 = slice(S * b, S * (b + 1))
                at_scr[sl3, :] = (av_l[b] / den_l[b]).astype(jnp.bfloat16)

        proj = jnp.dot(at_scr[...], wo_ref[l],
                       preferred_element_type=jnp.float32) + bo
        y = layer_norm(h + proj, g1, be1)

        ff = jnp.maximum(
            jnp.dot(y.astype(jnp.bfloat16), ffw1_ref[l],
                    preferred_element_type=jnp.float32) + ffb1_ref[l], 0.0)
        ff2 = jnp.dot(ff.astype(jnp.bfloat16), ffw2_ref[l],
                      preferred_element_type=jnp.float32) + b2
        h = layer_norm(y + ff2, g2, be2)

    out_ref[...] = h.reshape(bt, s, d)[:, 0, :]


def _const_spec(shape):
    n = len(shape)
    return pl.BlockSpec(shape, lambda g, _n=n: (0,) * _n)


def kernel(x, cls_token, wp_t, bp, pos_embedding, qkvo_w, layer_vec,
           ff_b1, ff_w1, ff_w2):
    B, seq_len, F = x.shape
    s = seq_len + 1
    scale = np.float32(1.0 / np.sqrt(HEAD_DIM))

    wq, wk, wv, wo = (qkvo_w[:, i] for i in range(4))
    bq, bk, bv = (layer_vec[:, i] for i in range(3))
    wqv = jnp.concatenate([wq * scale, wv], axis=2).astype(jnp.bfloat16)
    bqv = jnp.concatenate([bq * scale, bv], axis=1)[:, None, :]
    wk_bf = wk.astype(jnp.bfloat16)
    bkt = bk[:, :, None]
    wo_bf = wo.astype(jnp.bfloat16)
    vec6 = layer_vec[:, 3:9]
    wp_bf = wp_t.astype(jnp.bfloat16)
    ffw1_bf = ff_w1.astype(jnp.bfloat16)
    ffw2_bf = ff_w2.astype(jnp.bfloat16)
    pos = pos_embedding[0, :s, :]

    cls = jnp.broadcast_to(cls_token, (B, 1, F))
    x_cat = jnp.concatenate([cls, x], axis=1)
    B_pad = ((B + B_TILE - 1) // B_TILE) * B_TILE
    if B_pad != B:
        x_cat = jnp.concatenate(
            [x_cat, jnp.zeros((B_pad - B, s, F), x_cat.dtype)], axis=0)

    weights = [pos, wp_bf, bp, wqv, bqv, wk_bf, bkt, wo_bf, vec6,
               ffw1_bf, ff_b1, ffw2_bf]
    grid = (B_pad // B_TILE,)
    in_specs = ([pl.BlockSpec((B_TILE, s, F), lambda g: (g, 0, 0))]
                + [_const_spec(tuple(w.shape)) for w in weights])
    out_spec = pl.BlockSpec((B_TILE, D_MODEL), lambda g: (g, 0))

    m = B_TILE * s
    cls_hidden = pl.pallas_call(
        _enc_kernel,
        out_shape=jax.ShapeDtypeStruct((B_pad, D_MODEL), jnp.float32),
        grid=grid,
        in_specs=in_specs,
        out_specs=out_spec,
        scratch_shapes=[pltpu.VMEM((m, D_MODEL), jnp.bfloat16),
                        pltpu.VMEM((m, D_MODEL), jnp.bfloat16),
                        pltpu.VMEM((D_MODEL, m), jnp.bfloat16),
                        pltpu.VMEM((m, D_MODEL), jnp.bfloat16)],
        compiler_params=pltpu.CompilerParams(
            dimension_semantics=("parallel",)),
    )(x_cat, *weights)

    return cls_hidden[:B, :]


# speedup vs baseline: 8.3697x; 1.4878x over previous
import jax
import jax.numpy as jnp
import numpy as np
from jax.experimental import pallas as pl
from jax.experimental.pallas import tpu as pltpu

INPUT_DIM = 8
D_MODEL = 64
N_HEADS = 8
HEAD_DIM = D_MODEL // N_HEADS
NUM_LAYERS = 2
DIM_FF = 256
EPS = 1e-5
NEG_INF = -1e9
B_TILE = 64
S = 32
MH = N_HEADS * S


def _enc_kernel(x_ref, pos_ref, wp_ref, bp_ref, wqv_ref, bqv_ref, wk_ref,
                bkt_ref, wo_ref, vec_ref, ffw1_ref, ffb1_ref, ffw2_ref,
                out_ref, q_scr, v_scr, kt_scr, at_scr):
    bt, s, f = x_ref.shape
    m = bt * s
    d = D_MODEL

    x2 = x_ref[...].reshape(m, f)

    nz = jax.lax.dot_general(
        jnp.ones((1, f), jnp.float32), (x2 != 0.0).astype(jnp.float32),
        (((1,), (1,)), ((), ())), preferred_element_type=jnp.float32)
    col_j = jax.lax.broadcasted_iota(jnp.int32, (1, m), 1) % s
    pad_bias = jnp.where((nz == 0.0) & (col_j > 0), NEG_INF, 0.0)

    krow_h = jax.lax.broadcasted_iota(jnp.int32, (d, MH), 0) // HEAD_DIM
    kcol_h = jax.lax.broadcasted_iota(jnp.int32, (d, MH), 1) // S
    kmask_bf = jnp.where(krow_h == kcol_h, 1.0, 0.0).astype(jnp.bfloat16)
    vrow_h = jax.lax.broadcasted_iota(jnp.int32, (MH, d), 0) // S
    vcol_h = jax.lax.broadcasted_iota(jnp.int32, (MH, d), 1) // HEAD_DIM
    hm_bf = jnp.where(vrow_h == vcol_h, 1.0, 0.0).astype(jnp.bfloat16)

    h = (jnp.dot(x2.astype(jnp.bfloat16), wp_ref[...],
                 preferred_element_type=jnp.float32) + bp_ref[...])
    h = (h.reshape(bt, s, d) + pos_ref[...]).reshape(m, d)

    def layer_norm(z, g, b):
        mu = jnp.mean(z, axis=-1, keepdims=True)
        m2 = jnp.mean(jnp.square(z), axis=-1, keepdims=True)
        var = m2 - jnp.square(mu)
        return (z - mu) * jax.lax.rsqrt(var + EPS) * g + b

    for l in range(NUM_LAYERS - 1):
        vec = vec_ref[l]
        bo, g1, be1 = vec[0:1], vec[1:2], vec[2:3]
        b2, g2, be2 = vec[3:4], vec[4:5], vec[5:6]

        hb = h.astype(jnp.bfloat16)
        qv = (jnp.dot(hb, wqv_ref[l], preferred_element_type=jnp.float32)
              + bqv_ref[l])
        q_scr[...] = qv[:, :d].astype(jnp.bfloat16)
        v_scr[...] = qv[:, d:].astype(jnp.bfloat16)
        kt = jax.lax.dot_general(
            wk_ref[l], hb, (((0,), (1,)), ((), ())),
            preferred_element_type=jnp.float32) + bkt_ref[l]
        kt_scr[...] = kt.astype(jnp.bfloat16)

        logits_l = [None] * bt
        p_l = [None] * bt
        vrhs_l = [None] * bt
        den_l = [None] * bt
        av_l = [None] * bt
        pad_l = [None] * bt
        D1, D2, D3 = 6, 10, 13
        for i in range(bt + D3):
            if i < bt:
                b = i
                sl = slice(S * b, S * (b + 1))
                ktile = jnp.concatenate([kt_scr[:, sl]] * N_HEADS, axis=1)
                krhs_bf = ktile * kmask_bf
                logits_l[b] = jnp.dot(q_scr[sl, :], krhs_bf,
                                      preferred_element_type=jnp.float32)
                pad_l[b] = jnp.concatenate([pad_bias[:, sl]] * N_HEADS,
                                           axis=1)
                vtile = jnp.concatenate([v_scr[sl, :]] * N_HEADS, axis=0)
                vrhs_l[b] = vtile * hm_bf
            if D1 <= i < bt + D1:
                b = i - D1
                lg = logits_l[b]
                mx = jnp.max(lg, axis=-1, keepdims=True)
                p_l[b] = jnp.exp(lg + (pad_l[b] - mx)).astype(jnp.bfloat16)
            if D2 <= i < bt + D2:
                b = i - D2
                den_l[b] = jnp.dot(p_l[b], hm_bf,
                                   preferred_element_type=jnp.float32)
                av_l[b] = jnp.dot(p_l[b], vrhs_l[b],
                                  preferred_element_type=jnp.float32)
            if D3 <= i:
                b = i - D3
                sl3 = slice(S * b, S * (b + 1))
                at_scr[sl3, :] = (av_l[b] / den_l[b]).astype(jnp.bfloat16)

        proj = jnp.dot(at_scr[...], wo_ref[l],
                       preferred_element_type=jnp.float32) + bo
        y = layer_norm(h + proj, g1, be1)

        ff = jnp.maximum(
            jnp.dot(y.astype(jnp.bfloat16), ffw1_ref[l],
                    preferred_element_type=jnp.float32) + ffb1_ref[l], 0.0)
        ff2 = jnp.dot(ff.astype(jnp.bfloat16), ffw2_ref[l],
                      preferred_element_type=jnp.float32) + b2
        h = layer_norm(y + ff2, g2, be2)

    lz = NUM_LAYERS - 1
    vec = vec_ref[lz]
    bo, g1, be1 = vec[0:1], vec[1:2], vec[2:3]
    b2, g2, be2 = vec[3:4], vec[4:5], vec[5:6]

    hb = h.astype(jnp.bfloat16)
    v_scr[...] = (jnp.dot(hb, wqv_ref[lz][:, d:],
                          preferred_element_type=jnp.float32)
                  + bqv_ref[lz][:, d:]).astype(jnp.bfloat16)
    kt = jax.lax.dot_general(
        wk_ref[lz], hb, (((0,), (1,)), ((), ())),
        preferred_element_type=jnp.float32) + bkt_ref[lz]
    kt_scr[...] = kt.astype(jnp.bfloat16)

    hc = h.reshape(bt, s, d)[:, 0, :]
    qc = (jnp.dot(hc.astype(jnp.bfloat16), wqv_ref[lz][:, :d],
                  preferred_element_type=jnp.float32)
          + bqv_ref[lz][:, :d]).astype(jnp.bfloat16)

    ng = bt // 8
    qrow = jax.lax.broadcasted_iota(jnp.int32, (d, d), 0) // 8
    qcol = jax.lax.broadcasted_iota(jnp.int32, (d, d), 1) // HEAD_DIM
    hmq_bf = jnp.where(qrow == qcol, 1.0, 0.0).astype(jnp.bfloat16)
    hmq_f = jnp.where(qrow == qcol, 1.0, 0.0)
    crow = jax.lax.broadcasted_iota(jnp.int32, (d, 8 * S), 0) % 8
    ccol = jax.lax.broadcasted_iota(jnp.int32, (d, 8 * S), 1) // S
    cbias = jnp.where(crow == ccol, 0.0, NEG_INF)

    lg_l = [None] * ng
    p2_l = [None] * ng
    dn_l = [None] * ng
    av_l2 = [None] * ng
    outs2 = [None] * ng
    E1, E2, E3 = 2, 4, 5
    for i in range(ng + E3):
        if i < ng:
            g = i
            gsl = slice(8 * S * g, 8 * S * (g + 1))
            q8 = qc[8 * g:8 * (g + 1), :]
            qbd = jnp.concatenate([q8] * N_HEADS, axis=0) * hmq_bf
            lg = jnp.dot(qbd, kt_scr[:, gsl],
                         preferred_element_type=jnp.float32)
            lg_l[g] = lg + (cbias + pad_bias[:, gsl])
        if E1 <= i < ng + E1:
            g = i - E1
            lg = lg_l[g]
            mx = jnp.max(lg, axis=-1, keepdims=True)
            p = jnp.exp(lg - mx)
            dn_l[g] = jnp.sum(p, axis=-1, keepdims=True)
            p2_l[g] = p.astype(jnp.bfloat16)
        if E2 <= i < ng + E2:
            g = i - E2
            gsl = slice(8 * S * g, 8 * S * (g + 1))
            av_l2[g] = jnp.dot(p2_l[g], v_scr[gsl, :],
                               preferred_element_type=jnp.float32)
        if E3 <= i:
            g = i - E3
            r = (av_l2[g] / dn_l[g]) * hmq_f
            rc = r[0:8]
            for hh in range(1, N_HEADS):
                rc = rc + r[8 * hh:8 * (hh + 1)]
            outs2[g] = rc
    attn_c = jnp.concatenate(outs2, axis=0)

    proj2 = jnp.dot(attn_c.astype(jnp.bfloat16), wo_ref[lz],
                    preferred_element_type=jnp.float32) + bo
    y2 = layer_norm(hc + proj2, g1, be1)
    ffc = jnp.maximum(
        jnp.dot(y2.astype(jnp.bfloat16), ffw1_ref[lz],
                preferred_element_type=jnp.float32) + ffb1_ref[lz], 0.0)
    ff2c = jnp.dot(ffc.astype(jnp.bfloat16), ffw2_ref[lz],
                   preferred_element_type=jnp.float32) + b2
    out_ref[...] = layer_norm(y2 + ff2c, g2, be2)


def _const_spec(shape):
    n = len(shape)
    return pl.BlockSpec(shape, lambda g, _n=n: (0,) * _n)


def kernel(x, cls_token, wp_t, bp, pos_embedding, qkvo_w, layer_vec,
           ff_b1, ff_w1, ff_w2):
    B, seq_len, F = x.shape
    s = seq_len + 1
    scale = np.float32(1.0 / np.sqrt(HEAD_DIM))

    wq, wk, wv, wo = (qkvo_w[:, i] for i in range(4))
    bq, bk, bv = (layer_vec[:, i] for i in range(3))
    wqv = jnp.concatenate([wq * scale, wv], axis=2).astype(jnp.bfloat16)
    bqv = jnp.concatenate([bq * scale, bv], axis=1)[:, None, :]
    wk_bf = wk.astype(jnp.bfloat16)
    bkt = bk[:, :, None]
    wo_bf = wo.astype(jnp.bfloat16)
    vec6 = layer_vec[:, 3:9]
    wp_bf = wp_t.astype(jnp.bfloat16)
    ffw1_bf = ff_w1.astype(jnp.bfloat16)
    ffw2_bf = ff_w2.astype(jnp.bfloat16)
    pos = pos_embedding[0, :s, :]

    cls = jnp.broadcast_to(cls_token, (B, 1, F))
    x_cat = jnp.concatenate([cls, x], axis=1)
    B_pad = ((B + B_TILE - 1) // B_TILE) * B_TILE
    if B_pad != B:
        x_cat = jnp.concatenate(
            [x_cat, jnp.zeros((B_pad - B, s, F), x_cat.dtype)], axis=0)

    weights = [pos, wp_bf, bp, wqv, bqv, wk_bf, bkt, wo_bf, vec6,
               ffw1_bf, ff_b1, ffw2_bf]
    grid = (B_pad // B_TILE,)
    in_specs = ([pl.BlockSpec((B_TILE, s, F), lambda g: (g, 0, 0))]
                + [_const_spec(tuple(w.shape)) for w in weights])
    out_spec = pl.BlockSpec((B_TILE, D_MODEL), lambda g: (g, 0))

    m = B_TILE * s
    cls_hidden = pl.pallas_call(
        _enc_kernel,
        out_shape=jax.ShapeDtypeStruct((B_pad, D_MODEL), jnp.float32),
        grid=grid,
        in_specs=in_specs,
        out_specs=out_spec,
        scratch_shapes=[pltpu.VMEM((m, D_MODEL), jnp.bfloat16),
                        pltpu.VMEM((m, D_MODEL), jnp.bfloat16),
                        pltpu.VMEM((D_MODEL, m), jnp.bfloat16),
                        pltpu.VMEM((m, D_MODEL), jnp.bfloat16)],
        compiler_params=pltpu.CompilerParams(
            dimension_semantics=("parallel",)),
    )(x_cat, *weights)

    return cls_hidden[:B, :]


# speedup vs baseline: 9.1010x; 1.0874x over previous
import jax
import jax.numpy as jnp
import numpy as np
from jax.experimental import pallas as pl
from jax.experimental.pallas import tpu as pltpu

INPUT_DIM = 8
D_MODEL = 64
N_HEADS = 8
HEAD_DIM = D_MODEL // N_HEADS
NUM_LAYERS = 2
DIM_FF = 256
EPS = 1e-5
NEG_INF = -1e9
B_TILE = 128
S = 32
MH = N_HEADS * S


def _enc_kernel(x_ref, pos_ref, wp_ref, bp_ref, wqv_ref, bqv_ref, wk_ref,
                bkt_ref, wo_ref, vec_ref, ffw1_ref, ffb1_ref, ffw2_ref,
                out_ref, q_scr, v_scr, kt_scr, at_scr):
    bt, s, f = x_ref.shape
    m = bt * s
    d = D_MODEL

    x2 = x_ref[...].reshape(m, f)

    nz = jax.lax.dot_general(
        jnp.ones((1, f), jnp.float32), (x2 != 0.0).astype(jnp.float32),
        (((1,), (1,)), ((), ())), preferred_element_type=jnp.float32)
    col_j = jax.lax.broadcasted_iota(jnp.int32, (1, m), 1) % s
    pad_bias = jnp.where((nz == 0.0) & (col_j > 0), NEG_INF, 0.0)

    krow_h = jax.lax.broadcasted_iota(jnp.int32, (d, MH), 0) // HEAD_DIM
    kcol_h = jax.lax.broadcasted_iota(jnp.int32, (d, MH), 1) // S
    kmask_bf = jnp.where(krow_h == kcol_h, 1.0, 0.0).astype(jnp.bfloat16)
    vrow_h = jax.lax.broadcasted_iota(jnp.int32, (MH, d), 0) // S
    vcol_h = jax.lax.broadcasted_iota(jnp.int32, (MH, d), 1) // HEAD_DIM
    hm_bf = jnp.where(vrow_h == vcol_h, 1.0, 0.0).astype(jnp.bfloat16)

    h = (jnp.dot(x2.astype(jnp.bfloat16), wp_ref[...],
                 preferred_element_type=jnp.float32) + bp_ref[...])
    h = (h.reshape(bt, s, d) + pos_ref[...]).reshape(m, d)

    def layer_norm(z, g, b):
        mu = jnp.mean(z, axis=-1, keepdims=True)
        m2 = jnp.mean(jnp.square(z), axis=-1, keepdims=True)
        var = m2 - jnp.square(mu)
        return (z - mu) * jax.lax.rsqrt(var + EPS) * g + b

    for l in range(NUM_LAYERS - 1):
        vec = vec_ref[l]
        bo, g1, be1 = vec[0:1], vec[1:2], vec[2:3]
        b2, g2, be2 = vec[3:4], vec[4:5], vec[5:6]

        hb = h.astype(jnp.bfloat16)
        qv = (jnp.dot(hb, wqv_ref[l], preferred_element_type=jnp.float32)
              + bqv_ref[l])
        q_scr[...] = qv[:, :d].astype(jnp.bfloat16)
        v_scr[...] = qv[:, d:].astype(jnp.bfloat16)
        kt = jax.lax.dot_general(
            wk_ref[l], hb, (((0,), (1,)), ((), ())),
            preferred_element_type=jnp.float32) + bkt_ref[l]
        kt_scr[...] = kt.astype(jnp.bfloat16)

        logits_l = [None] * bt
        p_l = [None] * bt
        vrhs_l = [None] * bt
        den_l = [None] * bt
        av_l = [None] * bt
        pad_l = [None] * bt
        D1, D2, D3 = 6, 10, 13
        for i in range(bt + D3):
            if i < bt:
                b = i
                sl = slice(S * b, S * (b + 1))
                ktile = jnp.concatenate([kt_scr[:, sl]] * N_HEADS, axis=1)
                krhs_bf = ktile * kmask_bf
                logits_l[b] = jnp.dot(q_scr[sl, :], krhs_bf,
                                      preferred_element_type=jnp.float32)
                pad_l[b] = jnp.concatenate([pad_bias[:, sl]] * N_HEADS,
                                           axis=1)
                vtile = jnp.concatenate([v_scr[sl, :]] * N_HEADS, axis=0)
                vrhs_l[b] = vtile * hm_bf
            if D1 <= i < bt + D1:
                b = i - D1
                lg = logits_l[b]
                mx = jnp.max(lg, axis=-1, keepdims=True)
                p_l[b] = jnp.exp(lg + (pad_l[b] - mx)).astype(jnp.bfloat16)
            if D2 <= i < bt + D2:
                b = i - D2
                den_l[b] = jnp.dot(p_l[b], hm_bf,
                                   preferred_element_type=jnp.float32)
                av_l[b] = jnp.dot(p_l[b], vrhs_l[b],
                                  preferred_element_type=jnp.float32)
            if D3 <= i:
                b = i - D3
                sl3 = slice(S * b, S * (b + 1))
                at_scr[sl3, :] = (av_l[b] / den_l[b]).astype(jnp.bfloat16)

        proj = jnp.dot(at_scr[...], wo_ref[l],
                       preferred_element_type=jnp.float32) + bo
        y = layer_norm(h + proj, g1, be1)

        ff = jnp.maximum(
            jnp.dot(y.astype(jnp.bfloat16), ffw1_ref[l],
                    preferred_element_type=jnp.float32) + ffb1_ref[l], 0.0)
        ff2 = jnp.dot(ff.astype(jnp.bfloat16), ffw2_ref[l],
                      preferred_element_type=jnp.float32) + b2
        h = layer_norm(y + ff2, g2, be2)

    lz = NUM_LAYERS - 1
    vec = vec_ref[lz]
    bo, g1, be1 = vec[0:1], vec[1:2], vec[2:3]
    b2, g2, be2 = vec[3:4], vec[4:5], vec[5:6]

    hb = h.astype(jnp.bfloat16)
    v_scr[...] = (jnp.dot(hb, wqv_ref[lz][:, d:],
                          preferred_element_type=jnp.float32)
                  + bqv_ref[lz][:, d:]).astype(jnp.bfloat16)
    kt = jax.lax.dot_general(
        wk_ref[lz], hb, (((0,), (1,)), ((), ())),
        preferred_element_type=jnp.float32) + bkt_ref[lz]
    kt_scr[...] = kt.astype(jnp.bfloat16)

    hc = h.reshape(bt, s, d)[:, 0, :]
    qc = (jnp.dot(hc.astype(jnp.bfloat16), wqv_ref[lz][:, :d],
                  preferred_element_type=jnp.float32)
          + bqv_ref[lz][:, :d]).astype(jnp.bfloat16)

    ng = bt // 8
    qrow = jax.lax.broadcasted_iota(jnp.int32, (d, d), 0) // 8
    qcol = jax.lax.broadcasted_iota(jnp.int32, (d, d), 1) // HEAD_DIM
    hmq_bf = jnp.where(qrow == qcol, 1.0, 0.0).astype(jnp.bfloat16)
    hmq_f = jnp.where(qrow == qcol, 1.0, 0.0)
    crow = jax.lax.broadcasted_iota(jnp.int32, (d, 8 * S), 0) % 8
    ccol = jax.lax.broadcasted_iota(jnp.int32, (d, 8 * S), 1) // S
    cbias = jnp.where(crow == ccol, 0.0, NEG_INF)

    lg_l = [None] * ng
    p2_l = [None] * ng
    dn_l = [None] * ng
    av_l2 = [None] * ng
    outs2 = [None] * ng
    E1, E2, E3 = 3, 5, 7
    for i in range(ng + E3):
        if i < ng:
            g = i
            gsl = slice(8 * S * g, 8 * S * (g + 1))
            q8 = qc[8 * g:8 * (g + 1), :]
            qbd = jnp.concatenate([q8] * N_HEADS, axis=0) * hmq_bf
            lg = jnp.dot(qbd, kt_scr[:, gsl],
                         preferred_element_type=jnp.float32)
            lg_l[g] = lg + (cbias + pad_bias[:, gsl])
        if E1 <= i < ng + E1:
            g = i - E1
            lg = lg_l[g]
            mx = jnp.max(lg, axis=-1, keepdims=True)
            p = jnp.exp(lg - mx)
            dn_l[g] = jnp.sum(p, axis=-1, keepdims=True)
            p2_l[g] = p.astype(jnp.bfloat16)
        if E2 <= i < ng + E2:
            g = i - E2
            gsl = slice(8 * S * g, 8 * S * (g + 1))
            av_l2[g] = jnp.dot(p2_l[g], v_scr[gsl, :],
                               preferred_element_type=jnp.float32)
        if E3 <= i:
            g = i - E3
            r = (av_l2[g] / dn_l[g]) * hmq_f
            rc = r[0:8]
            for hh in range(1, N_HEADS):
                rc = rc + r[8 * hh:8 * (hh + 1)]
            outs2[g] = rc
    attn_c = jnp.concatenate(outs2, axis=0)

    proj2 = jnp.dot(attn_c.astype(jnp.bfloat16), wo_ref[lz],
                    preferred_element_type=jnp.float32) + bo
    y2 = layer_norm(hc + proj2, g1, be1)
    ffc = jnp.maximum(
        jnp.dot(y2.astype(jnp.bfloat16), ffw1_ref[lz],
                preferred_element_type=jnp.float32) + ffb1_ref[lz], 0.0)
    ff2c = jnp.dot(ffc.astype(jnp.bfloat16), ffw2_ref[lz],
                   preferred_element_type=jnp.float32) + b2
    out_ref[...] = layer_norm(y2 + ff2c, g2, be2)


def _const_spec(shape):
    n = len(shape)
    return pl.BlockSpec(shape, lambda g, _n=n: (0,) * _n)


def kernel(x, cls_token, wp_t, bp, pos_embedding, qkvo_w, layer_vec,
           ff_b1, ff_w1, ff_w2):
    B, seq_len, F = x.shape
    s = seq_len + 1
    scale = np.float32(1.0 / np.sqrt(HEAD_DIM))

    wq, wk, wv, wo = (qkvo_w[:, i] for i in range(4))
    bq, bk, bv = (layer_vec[:, i] for i in range(3))
    wqv = jnp.concatenate([wq * scale, wv], axis=2).astype(jnp.bfloat16)
    bqv = jnp.concatenate([bq * scale, bv], axis=1)[:, None, :]
    wk_bf = wk.astype(jnp.bfloat16)
    bkt = bk[:, :, None]
    wo_bf = wo.astype(jnp.bfloat16)
    vec6 = layer_vec[:, 3:9]
    wp_bf = wp_t.astype(jnp.bfloat16)
    ffw1_bf = ff_w1.astype(jnp.bfloat16)
    ffw2_bf = ff_w2.astype(jnp.bfloat16)
    pos = pos_embedding[0, :s, :]

    cls = jnp.broadcast_to(cls_token, (B, 1, F))
    x_cat = jnp.concatenate([cls, x], axis=1)
    B_pad = ((B + B_TILE - 1) // B_TILE) * B_TILE
    if B_pad != B:
        x_cat = jnp.concatenate(
            [x_cat, jnp.zeros((B_pad - B, s, F), x_cat.dtype)], axis=0)

    weights = [pos, wp_bf, bp, wqv, bqv, wk_bf, bkt, wo_bf, vec6,
               ffw1_bf, ff_b1, ffw2_bf]
    grid = (B_pad // B_TILE,)
    in_specs = ([pl.BlockSpec((B_TILE, s, F), lambda g: (g, 0, 0))]
                + [_const_spec(tuple(w.shape)) for w in weights])
    out_spec = pl.BlockSpec((B_TILE, D_MODEL), lambda g: (g, 0))

    m = B_TILE * s
    cls_hidden = pl.pallas_call(
        _enc_kernel,
        out_shape=jax.ShapeDtypeStruct((B_pad, D_MODEL), jnp.float32),
        grid=grid,
        in_specs=in_specs,
        out_specs=out_spec,
        scratch_shapes=[pltpu.VMEM((m, D_MODEL), jnp.bfloat16),
                        pltpu.VMEM((m, D_MODEL), jnp.bfloat16),
                        pltpu.VMEM((D_MODEL, m), jnp.bfloat16),
                        pltpu.VMEM((m, D_MODEL), jnp.bfloat16)],
        compiler_params=pltpu.CompilerParams(
            dimension_semantics=("parallel",)),
    )(x_cat, *weights)

    return cls_hidden[:B, :]
